# baseline (speedup 1.0000x reference)
import functools
import jax
import jax.numpy as jnp
from jax import lax
from jax.experimental import pallas as pl
from jax.experimental.pallas import tpu as pltpu
from jax.experimental.pallas import tpu_sc as plsc

N = 100000
E = 1600000
LANE = 128
ROWS = 12800
PAD_E = ROWS * LANE
NC, NS = 2, 16
DUMP = N
NACC = 100096
ZROWS = 256
STRIPE = NACC // NS
BLKE = 512

_f32 = jnp.float32


def _fill(buf, val, n):
  def body(i, carry):
    buf[i] = jnp.full((16,), val, _f32)
    return carry
  lax.fori_loop(0, n, body, None)


def _zero_stripe(acc, s, zbuf):
  base = s * STRIPE
  nfull = STRIPE // ZROWS
  for i in range(nfull):
    pltpu.sync_copy(zbuf, acc.at[pl.ds(base + i * ZROWS, ZROWS)])
  tail = STRIPE - nfull * ZROWS
  if tail:
    pltpu.sync_copy(zbuf.at[pl.ds(0, tail)],
                    acc.at[pl.ds(base + nfull * ZROWS, tail)])


def _deg_body(dst_hbm, deg0_hbm, deg1_hbm, acc, zbuf, ones, idxb, sem):
  c = lax.axis_index("c")
  s = lax.axis_index("s")
  _fill(zbuf, 0.0, ZROWS)
  _fill(ones, 1.0, BLKE)
  _zero_stripe(acc, s, zbuf)
  plsc.subcore_barrier()
  edges_per_tile = PAD_E // (NC * NS)
  base_e = (c * NS + s) * edges_per_tile

  def blk(b, carry):
    e0 = pl.multiple_of(base_e + b * BLKE, BLKE)
    pltpu.sync_copy(dst_hbm.at[pl.ds(e0, BLKE)], idxb)
    pltpu.sync_copy(ones, acc.at[idxb], add=True)
    return carry

  lax.fori_loop(0, edges_per_tile // BLKE, blk, None)
  plsc.subcore_barrier()
  for cv, out in ((0, deg0_hbm), (1, deg1_hbm)):
    @pl.when(c == cv)
    def _copyout(out=out):
      pltpu.sync_copy(acc.at[pl.ds(s * STRIPE, STRIPE)],
                      out.at[pl.ds(s * STRIPE, STRIPE)])


@functools.lru_cache(maxsize=None)
def _deg():
  mesh = plsc.VectorSubcoreMesh(core_axis_name="c", subcore_axis_name="s")
  return functools.partial(
      pl.kernel,
      out_type=[jax.ShapeDtypeStruct((NACC, 16), _f32)] * 2,
      mesh=mesh,
      compiler_params=pltpu.CompilerParams(use_tc_tiling_on_sc=False),
      scratch_types=[
          pltpu.VMEM_SHARED((NACC, 16), _f32),
          pltpu.VMEM((ZROWS, 16), _f32),
          pltpu.VMEM((BLKE, 16), _f32),
          pltpu.VMEM((BLKE,), jnp.int32),
          pltpu.SemaphoreType.DMA,
      ],
  )(_deg_body)


def _scatter_body(e3_hbm, z0, z1, z2, z3,
                  s0, s1, s2, s3, acc, zbuf,
                  idx0, rows0, idx1, rows1, sem, isem):
  c = lax.axis_index("c")
  s = lax.axis_index("s")
  _fill(zbuf, 0.0, ZROWS)
  zs = (z0, z1, z2, z3)
  outs = (s0, s1, s2, s3)
  bufs = ((idx0, rows0), (idx1, rows1))
  blocks_per_tile = PAD_E // (NS * BLKE)

  for cv in (0, 1):
    @pl.when(c == cv)
    def _core(cv=cv):
      for r in (0, 1):
        ch = 2 * cv + r
        zc, oc = zs[ch], outs[ch]
        _zero_stripe(acc, s, zbuf)
        plsc.subcore_barrier()

        last_blk = PAD_E // BLKE - 1

        def load_idx(g, p):
          blk = jnp.minimum(s * blocks_per_tile + 2 * g + p, last_blk)
          return pltpu.async_copy(e3_hbm.at[blk], bufs[p][0], isem)

        load_idx(0, 0).wait()
        load_idx(0, 1).wait()

        def grp(g, carry, zc=zc):
          descs = [pltpu.async_copy(zc.at[bufs[p][0].at[0]], bufs[p][1], sem)
                   for p in (0, 1)]
          pre = []
          for p in (0, 1):
            idx, rows = bufs[p]
            descs[p].wait()
            pltpu.sync_copy(rows, acc.at[idx.at[1]], add=True)
            pre.append(load_idx(g + 1, p))
          for d in pre:
            d.wait()
          return carry

        lax.fori_loop(0, blocks_per_tile // 2, grp, None)
        plsc.subcore_barrier()
        pltpu.sync_copy(acc.at[pl.ds(s * STRIPE, STRIPE)],
                        oc.at[pl.ds(s * STRIPE, STRIPE)])
        plsc.subcore_barrier()


@functools.lru_cache(maxsize=None)
def _scatter():
  mesh = plsc.VectorSubcoreMesh(core_axis_name="c", subcore_axis_name="s")
  return functools.partial(
      pl.kernel,
      out_type=[jax.ShapeDtypeStruct((NACC, 16), _f32)] * 4,
      mesh=mesh,
      compiler_params=pltpu.CompilerParams(use_tc_tiling_on_sc=False),
      scratch_types=[
          pltpu.VMEM_SHARED((NACC, 16), _f32),
          pltpu.VMEM((ZROWS, 16), _f32),
          pltpu.VMEM((2, BLKE), jnp.int32),
          pltpu.VMEM((BLKE, 16), _f32),
          pltpu.VMEM((2, BLKE), jnp.int32),
          pltpu.VMEM((BLKE, 16), _f32),
          pltpu.SemaphoreType.DMA,
          pltpu.SemaphoreType.DMA,
      ],
  )(_scatter_body)


def _gather_body(table_hbm, ids_hbm, out_hbm, idx4, rows, sem):
  c = lax.axis_index("c")
  s = lax.axis_index("s")
  wid = s * NC + c
  pltpu.sync_copy(ids_hbm.at[pl.ds(wid * 4, 4)], idx4)
  for j in range(4):
    pltpu.async_copy(table_hbm.at[idx4.at[j]], rows, sem).wait()
    pltpu.sync_copy(rows, out_hbm.at[pl.ds(wid * 512 + j * LANE, LANE)])


@functools.lru_cache(maxsize=None)
def _gather():
  mesh = plsc.VectorSubcoreMesh(core_axis_name="c", subcore_axis_name="s")
  return functools.partial(
      pl.kernel,
      out_type=jax.ShapeDtypeStruct((16384, 64), _f32),
      mesh=mesh,
      compiler_params=pltpu.CompilerParams(use_tc_tiling_on_sc=False),
      scratch_types=[
          pltpu.VMEM((4, LANE), jnp.int32),
          pltpu.VMEM((LANE, 64), _f32),
          pltpu.SemaphoreType.DMA,
      ],
  )(_gather_body)


_BM = 1000


def _dis(d0, d1):
  deg = d0[:, 0:1] + d1[:, 0:1] + 1.0
  return lax.rsqrt(deg)


def _k1_body(d0, d1, emb, w1, o0, o1, o2, o3):
  dis = _dis(d0[...], d1[...])
  z = jnp.dot(emb[...], w1[...], preferred_element_type=_f32) * dis
  o0[...] = z[:, 0:16]
  o1[...] = z[:, 16:32]
  o2[...] = z[:, 32:48]
  o3[...] = z[:, 48:64]


def _k2_body(d0, d1, s0, s1, s2, s3, z0, z1, z2, z3, b1, w2,
             o0, o1, o2, o3):
  dis = _dis(d0[...], d1[...])
  ssum = jnp.concatenate([s0[...] + z0[...], s1[...] + z1[...],
                          s2[...] + z2[...], s3[...] + z3[...]], axis=1)
  x1 = jax.nn.relu(dis * ssum + b1[...])
  z = jnp.dot(x1, w2[...], preferred_element_type=_f32) * dis
  o0[...] = z[:, 0:16]
  o1[...] = z[:, 16:32]
  o2[...] = z[:, 32:48]
  o3[...] = z[:, 48:64]


def _k3_body(d0, d1, s0, s1, s2, s3, z0, z1, z2, z3, b2, out):
  dis = _dis(d0[...], d1[...])
  ssum = jnp.concatenate([s0[...] + z0[...], s1[...] + z1[...],
                          s2[...] + z2[...], s3[...] + z3[...]], axis=1)
  out[...] = dis * ssum + b2[...]


def _k4_body(drug, lab, wl1, bl1, wl2, bl2, wf1, bf1, wf2, bf2, out):
  lab_h = jax.nn.relu(
      jnp.dot(lab[...], wl1[...], preferred_element_type=_f32) + bl1[...])
  lab_e = jnp.dot(lab_h, wl2[...], preferred_element_type=_f32) + bl2[...]
  comb = jnp.concatenate([drug[...], lab_e], axis=1)
  h = jax.nn.relu(
      jnp.dot(comb, wf1[...], preferred_element_type=_f32) + bf1[...])
  o = jnp.dot(h, wf2[...], preferred_element_type=_f32) + bf2[...]
  out[...] = jax.nn.sigmoid(o)


def _row_spec(cols):
  return pl.BlockSpec((_BM, cols), lambda i: (i, 0))


def _full_spec(r, c):
  return pl.BlockSpec((r, c), lambda i: (0, 0))


def _k1(d0, d1, emb, w1):
  return pl.pallas_call(
      _k1_body,
      grid=(N // _BM,),
      in_specs=[_row_spec(16), _row_spec(16), _row_spec(32),
                _full_spec(32, 64)],
      out_specs=[_row_spec(16)] * 4,
      out_shape=[jax.ShapeDtypeStruct((N, 16), _f32)] * 4,
  )(d0, d1, emb, w1)


def _k2(d0, d1, scs, zcs, b1, w2):
  return pl.pallas_call(
      _k2_body,
      grid=(N // _BM,),
      in_specs=[_row_spec(16)] * 10 + [_full_spec(1, 64), _full_spec(64, 64)],
      out_specs=[_row_spec(16)] * 4,
      out_shape=[jax.ShapeDtypeStruct((N, 16), _f32)] * 4,
  )(d0, d1, *scs, *zcs, b1, w2)


def _k3(d0, d1, scs, zcs, b2):
  return pl.pallas_call(
      _k3_body,
      grid=(N // _BM,),
      in_specs=[_row_spec(16)] * 10 + [_full_spec(1, 64)],
      out_specs=pl.BlockSpec((_BM, 64), lambda i: (i, 0)),
      out_shape=jax.ShapeDtypeStruct((N, 64), _f32),
  )(d0, d1, *scs, *zcs, b2)


def _k4(drug, lab, wl1, bl1, wl2, bl2, wf1, bf1, wf2, bf2):
  bm = 2048
  return pl.pallas_call(
      _k4_body,
      grid=(16384 // bm,),
      in_specs=[
          pl.BlockSpec((bm, 64), lambda i: (i, 0)),
          pl.BlockSpec((bm, 9), lambda i: (i, 0)),
          _full_spec(9, 32), _full_spec(1, 32),
          _full_spec(32, 32), _full_spec(1, 32),
          _full_spec(96, 64), _full_spec(1, 64),
          _full_spec(64, 1), _full_spec(1, 1),
      ],
      out_specs=pl.BlockSpec((bm, 1), lambda i: (i, 0)),
      out_shape=jax.ShapeDtypeStruct((16384, 1), _f32),
  )(drug, lab, wl1, bl1, wl2, bl2, wf1, bf1, wf2, bf2)


def kernel(edge_index, drug_ids, lab_features, emb, W1, b1, W2, b2,
           Wl1, bl1, Wl2, bl2, Wf1, bf1, Wf2, bf2):
  pad = PAD_E - E
  srcp = jnp.concatenate([edge_index[0], jnp.zeros((pad,), jnp.int32)])
  dstp = jnp.concatenate([edge_index[1], jnp.full((pad,), DUMP, jnp.int32)])
  e3 = jnp.stack([srcp.reshape(-1, BLKE), dstp.reshape(-1, BLKE)], axis=1)
  ids2 = drug_ids.reshape(LANE, LANE)

  d0, d1 = _deg()(dstp)
  zc1 = _k1(d0, d1, emb, W1)
  sc1 = _scatter()(e3, *zc1)
  zc2 = _k2(d0, d1, sc1, zc1, b1.reshape(1, 64), W2)
  sc2 = _scatter()(e3, *zc2)
  out2 = _k3(d0, d1, sc2, zc2, b2.reshape(1, 64))
  drug = _gather()(out2, ids2)
  o = _k4(drug, lab_features, Wl1, bl1.reshape(1, 32), Wl2, bl2.reshape(1, 32),
          Wf1, bf1.reshape(1, 64), Wf2, bf2.reshape(1, 1))
  return o.reshape(-1)


# speedup vs baseline: 1.0944x; 1.0944x over previous
import functools
import jax
import jax.numpy as jnp
from jax import lax
from jax.experimental import pallas as pl
from jax.experimental.pallas import tpu as pltpu
from jax.experimental.pallas import tpu_sc as plsc

N = 100000
E = 1600000
LANE = 128
ROWS = 12800
PAD_E = ROWS * LANE
NC, NS = 2, 16
DUMP = N
NACC = 100096
ZROWS = 256
STRIPE = NACC // NS
BLKE = 512

_f32 = jnp.float32


def _fill(buf, val, n):
  def body(i, carry):
    buf[i] = jnp.full((16,), val, _f32)
    return carry
  lax.fori_loop(0, n, body, None)


def _zero_stripe(acc, s, zbuf):
  base = s * STRIPE
  nfull = STRIPE // ZROWS
  for i in range(nfull):
    pltpu.sync_copy(zbuf, acc.at[pl.ds(base + i * ZROWS, ZROWS)])
  tail = STRIPE - nfull * ZROWS
  if tail:
    pltpu.sync_copy(zbuf.at[pl.ds(0, tail)],
                    acc.at[pl.ds(base + nfull * ZROWS, tail)])


def _deg_body(dst_hbm, deg0_hbm, deg1_hbm, acc, zbuf, ones, idxb, sem):
  c = lax.axis_index("c")
  s = lax.axis_index("s")
  _fill(zbuf, 0.0, ZROWS)
  _fill(ones, 1.0, BLKE)
  _zero_stripe(acc, s, zbuf)
  plsc.subcore_barrier()
  edges_per_tile = PAD_E // (NC * NS)
  base_e = (c * NS + s) * edges_per_tile

  def blk(b, carry):
    e0 = pl.multiple_of(base_e + b * BLKE, BLKE)
    pltpu.sync_copy(dst_hbm.at[pl.ds(e0, BLKE)], idxb)
    pltpu.sync_copy(ones, acc.at[idxb], add=True)
    return carry

  lax.fori_loop(0, edges_per_tile // BLKE, blk, None)
  plsc.subcore_barrier()
  for cv, out in ((0, deg0_hbm), (1, deg1_hbm)):
    @pl.when(c == cv)
    def _copyout(out=out):
      pltpu.sync_copy(acc.at[pl.ds(s * STRIPE, STRIPE)],
                      out.at[pl.ds(s * STRIPE, STRIPE)])


@functools.lru_cache(maxsize=None)
def _deg():
  mesh = plsc.VectorSubcoreMesh(core_axis_name="c", subcore_axis_name="s")
  return functools.partial(
      pl.kernel,
      out_type=[jax.ShapeDtypeStruct((NACC, 16), _f32)] * 2,
      mesh=mesh,
      compiler_params=pltpu.CompilerParams(use_tc_tiling_on_sc=False),
      scratch_types=[
          pltpu.VMEM_SHARED((NACC, 16), _f32),
          pltpu.VMEM((ZROWS, 16), _f32),
          pltpu.VMEM((BLKE, 16), _f32),
          pltpu.VMEM((BLKE,), jnp.int32),
          pltpu.SemaphoreType.DMA,
      ],
  )(_deg_body)


def _scatter_body(e3_hbm, z0, z1, z2, z3,
                  s0, s1, s2, s3, acc, zbuf,
                  idx0, rows0, idx1, rows1, sem, isem):
  c = lax.axis_index("c")
  s = lax.axis_index("s")
  _fill(zbuf, 0.0, ZROWS)
  zs = (z0, z1, z2, z3)
  outs = (s0, s1, s2, s3)
  bufs = ((idx0, rows0), (idx1, rows1))
  blocks_per_tile = PAD_E // (NS * BLKE)

  for cv in (0, 1):
    @pl.when(c == cv)
    def _core(cv=cv):
      for r in (0, 1):
        ch = 2 * cv + r
        zc, oc = zs[ch], outs[ch]
        _zero_stripe(acc, s, zbuf)
        plsc.subcore_barrier()

        last_blk = PAD_E // BLKE - 1

        def load_idx(g, p):
          blk = jnp.minimum(s * blocks_per_tile + 2 * g + p, last_blk)
          return pltpu.async_copy(e3_hbm.at[blk], bufs[p][0], isem)

        load_idx(0, 0).wait()
        load_idx(0, 1).wait()

        def grp(g, carry, zc=zc):
          descs = [pltpu.async_copy(zc.at[bufs[p][0].at[0]], bufs[p][1], sem)
                   for p in (0, 1)]
          pre = []
          for p in (0, 1):
            idx, rows = bufs[p]
            descs[p].wait()
            pltpu.sync_copy(rows, acc.at[idx.at[1]], add=True)
            pre.append(load_idx(g + 1, p))
          for d in pre:
            d.wait()
          return carry

        lax.fori_loop(0, blocks_per_tile // 2, grp, None)
        plsc.subcore_barrier()
        pltpu.sync_copy(acc.at[pl.ds(s * STRIPE, STRIPE)],
                        oc.at[pl.ds(s * STRIPE, STRIPE)])
        plsc.subcore_barrier()


@functools.lru_cache(maxsize=None)
def _scatter():
  mesh = plsc.VectorSubcoreMesh(core_axis_name="c", subcore_axis_name="s")
  return functools.partial(
      pl.kernel,
      out_type=[jax.ShapeDtypeStruct((NACC, 16), _f32)] * 4,
      mesh=mesh,
      compiler_params=pltpu.CompilerParams(use_tc_tiling_on_sc=False),
      scratch_types=[
          pltpu.VMEM_SHARED((NACC, 16), _f32),
          pltpu.VMEM((ZROWS, 16), _f32),
          pltpu.VMEM((2, BLKE), jnp.int32),
          pltpu.VMEM((BLKE, 16), _f32),
          pltpu.VMEM((2, BLKE), jnp.int32),
          pltpu.VMEM((BLKE, 16), _f32),
          pltpu.SemaphoreType.DMA,
          pltpu.SemaphoreType.DMA,
      ],
  )(_scatter_body)


def _gather_body(a0, a1, a2, a3, a4, a5, a6, a7, a8, a9, ids_hbm,
                 o0, o1, o2, o3, o4, o5, o6, o7, o8, o9, idxb, rows, sem):
  c = lax.axis_index("c")
  s = lax.axis_index("s")
  wid = s * NC + c
  base = pl.multiple_of(wid * 512, 8)
  pltpu.sync_copy(ids_hbm.at[pl.ds(base, 512)], idxb)
  for a, o in ((a0, o0), (a1, o1), (a2, o2), (a3, o3), (a4, o4),
               (a5, o5), (a6, o6), (a7, o7), (a8, o8), (a9, o9)):
    pltpu.async_copy(a.at[idxb], rows, sem).wait()
    pltpu.sync_copy(rows, o.at[pl.ds(base, 512)])


@functools.lru_cache(maxsize=None)
def _gather():
  mesh = plsc.VectorSubcoreMesh(core_axis_name="c", subcore_axis_name="s")
  return functools.partial(
      pl.kernel,
      out_type=[jax.ShapeDtypeStruct((16384, 16), _f32)] * 10,
      mesh=mesh,
      compiler_params=pltpu.CompilerParams(use_tc_tiling_on_sc=False),
      scratch_types=[
          pltpu.VMEM((512,), jnp.int32),
          pltpu.VMEM((512, 16), _f32),
          pltpu.SemaphoreType.DMA,
      ],
  )(_gather_body)


_BM = 1000


def _dis(d0, d1):
  deg = d0[:, 0:1] + d1[:, 0:1] + 1.0
  return lax.rsqrt(deg)


def _k1_body(d0, d1, emb, w1, o0, o1, o2, o3):
  dis = _dis(d0[...], d1[...])
  z = jnp.dot(emb[...], w1[...], preferred_element_type=_f32) * dis
  o0[...] = z[:, 0:16]
  o1[...] = z[:, 16:32]
  o2[...] = z[:, 32:48]
  o3[...] = z[:, 48:64]


def _k2_body(d0, d1, s0, s1, s2, s3, z0, z1, z2, z3, b1, w2,
             o0, o1, o2, o3):
  dis = _dis(d0[...], d1[...])
  ssum = jnp.concatenate([s0[...] + z0[...], s1[...] + z1[...],
                          s2[...] + z2[...], s3[...] + z3[...]], axis=1)
  x1 = jax.nn.relu(dis * ssum + b1[...])
  z = jnp.dot(x1, w2[...], preferred_element_type=_f32) * dis
  o0[...] = z[:, 0:16]
  o1[...] = z[:, 16:32]
  o2[...] = z[:, 32:48]
  o3[...] = z[:, 48:64]


def _k3_body(d0, d1, s0, s1, s2, s3, z0, z1, z2, z3, b2, out):
  dis = _dis(d0[...], d1[...])
  ssum = jnp.concatenate([s0[...] + z0[...], s1[...] + z1[...],
                          s2[...] + z2[...], s3[...] + z3[...]], axis=1)
  out[...] = dis * ssum + b2[...]


def _k4_body(d0, d1, s0, s1, s2, s3, z0, z1, z2, z3,
             lab, b2, wl1, bl1, wl2, bl2, wf1, bf1, wf2, bf2, out):
  dis = _dis(d0[...], d1[...])
  ssum = jnp.concatenate([s0[...] + z0[...], s1[...] + z1[...],
                          s2[...] + z2[...], s3[...] + z3[...]], axis=1)
  drug = dis * ssum + b2[...]
  lab_h = jax.nn.relu(
      jnp.dot(lab[...], wl1[...], preferred_element_type=_f32) + bl1[...])
  lab_e = jnp.dot(lab_h, wl2[...], preferred_element_type=_f32) + bl2[...]
  comb = jnp.concatenate([drug, lab_e], axis=1)
  h = jax.nn.relu(
      jnp.dot(comb, wf1[...], preferred_element_type=_f32) + bf1[...])
  o = jnp.dot(h, wf2[...], preferred_element_type=_f32) + bf2[...]
  out[...] = jax.nn.sigmoid(o)


def _row_spec(cols):
  return pl.BlockSpec((_BM, cols), lambda i: (i, 0))


def _full_spec(r, c):
  return pl.BlockSpec((r, c), lambda i: (0, 0))


def _k1(d0, d1, emb, w1):
  return pl.pallas_call(
      _k1_body,
      grid=(N // _BM,),
      in_specs=[_row_spec(16), _row_spec(16), _row_spec(32),
                _full_spec(32, 64)],
      out_specs=[_row_spec(16)] * 4,
      out_shape=[jax.ShapeDtypeStruct((N, 16), _f32)] * 4,
  )(d0, d1, emb, w1)


def _k2(d0, d1, scs, zcs, b1, w2):
  return pl.pallas_call(
      _k2_body,
      grid=(N // _BM,),
      in_specs=[_row_spec(16)] * 10 + [_full_spec(1, 64), _full_spec(64, 64)],
      out_specs=[_row_spec(16)] * 4,
      out_shape=[jax.ShapeDtypeStruct((N, 16), _f32)] * 4,
  )(d0, d1, *scs, *zcs, b1, w2)


def _k3(d0, d1, scs, zcs, b2):
  return pl.pallas_call(
      _k3_body,
      grid=(N // _BM,),
      in_specs=[_row_spec(16)] * 10 + [_full_spec(1, 64)],
      out_specs=pl.BlockSpec((_BM, 64), lambda i: (i, 0)),
      out_shape=jax.ShapeDtypeStruct((N, 64), _f32),
  )(d0, d1, *scs, *zcs, b2)


def _k4(g10, lab, b2, wl1, bl1, wl2, bl2, wf1, bf1, wf2, bf2):
  bm = 2048
  gspec = pl.BlockSpec((bm, 16), lambda i: (i, 0))
  return pl.pallas_call(
      _k4_body,
      grid=(16384 // bm,),
      in_specs=[gspec] * 10 + [
          pl.BlockSpec((bm, 9), lambda i: (i, 0)),
          _full_spec(1, 64),
          _full_spec(9, 32), _full_spec(1, 32),
          _full_spec(32, 32), _full_spec(1, 32),
          _full_spec(96, 64), _full_spec(1, 64),
          _full_spec(64, 1), _full_spec(1, 1),
      ],
      out_specs=pl.BlockSpec((bm, 1), lambda i: (i, 0)),
      out_shape=jax.ShapeDtypeStruct((16384, 1), _f32),
  )(*g10, lab, b2, wl1, bl1, wl2, bl2, wf1, bf1, wf2, bf2)


def kernel(edge_index, drug_ids, lab_features, emb, W1, b1, W2, b2,
           Wl1, bl1, Wl2, bl2, Wf1, bf1, Wf2, bf2):
  pad = PAD_E - E
  srcp = jnp.concatenate([edge_index[0], jnp.zeros((pad,), jnp.int32)])
  dstp = jnp.concatenate([edge_index[1], jnp.full((pad,), DUMP, jnp.int32)])
  e3 = jnp.stack([srcp.reshape(-1, BLKE), dstp.reshape(-1, BLKE)], axis=1)

  d0, d1 = _deg()(dstp)
  zc1 = _k1(d0, d1, emb, W1)
  sc1 = _scatter()(e3, *zc1)
  zc2 = _k2(d0, d1, sc1, zc1, b1.reshape(1, 64), W2)
  sc2 = _scatter()(e3, *zc2)
  g10 = _gather()(d0, d1, *sc2, *zc2, drug_ids)
  o = _k4(g10, lab_features, b2.reshape(1, 64),
          Wl1, bl1.reshape(1, 32), Wl2, bl2.reshape(1, 32),
          Wf1, bf1.reshape(1, 64), Wf2, bf2.reshape(1, 1))
  return o.reshape(-1)


# speedup vs baseline: 1.0944x; 1.0000x over previous
import functools
import jax
import jax.numpy as jnp
from jax import lax
from jax.experimental import pallas as pl
from jax.experimental.pallas import tpu as pltpu
from jax.experimental.pallas import tpu_sc as plsc

N = 100000
E = 1600000
LANE = 128
ROWS = 12800
PAD_E = ROWS * LANE
NC, NS = 2, 16
DUMP = N
NACC = 100096
ZROWS = 256
STRIPE = NACC // NS
BLKE = 512

_f32 = jnp.float32


def _fill(buf, val, n):
  def body(i, carry):
    buf[i] = jnp.full((16,), val, _f32)
    return carry
  lax.fori_loop(0, n, body, None)


def _zero_stripe(acc, s, zbuf):
  base = s * STRIPE
  nfull = STRIPE // ZROWS
  for i in range(nfull):
    pltpu.sync_copy(zbuf, acc.at[pl.ds(base + i * ZROWS, ZROWS)])
  tail = STRIPE - nfull * ZROWS
  if tail:
    pltpu.sync_copy(zbuf.at[pl.ds(0, tail)],
                    acc.at[pl.ds(base + nfull * ZROWS, tail)])


def _deg_body(dst_hbm, deg0_hbm, deg1_hbm, acc, zbuf, ones, idxb, sem):
  c = lax.axis_index("c")
  s = lax.axis_index("s")
  _fill(zbuf, 0.0, ZROWS)
  _fill(ones, 1.0, BLKE)
  _zero_stripe(acc, s, zbuf)
  plsc.subcore_barrier()
  edges_per_tile = PAD_E // (NC * NS)
  base_e = (c * NS + s) * edges_per_tile

  def blk(b, carry):
    e0 = pl.multiple_of(base_e + b * BLKE, BLKE)
    pltpu.sync_copy(dst_hbm.at[pl.ds(e0, BLKE)], idxb)
    pltpu.sync_copy(ones, acc.at[idxb], add=True)
    return carry

  lax.fori_loop(0, edges_per_tile // BLKE, blk, None)
  plsc.subcore_barrier()
  for cv, out in ((0, deg0_hbm), (1, deg1_hbm)):
    @pl.when(c == cv)
    def _copyout(out=out):
      pltpu.sync_copy(acc.at[pl.ds(s * STRIPE, STRIPE)],
                      out.at[pl.ds(s * STRIPE, STRIPE)])


@functools.lru_cache(maxsize=None)
def _deg():
  mesh = plsc.VectorSubcoreMesh(core_axis_name="c", subcore_axis_name="s")
  return functools.partial(
      pl.kernel,
      out_type=[jax.ShapeDtypeStruct((NACC, 16), _f32)] * 2,
      mesh=mesh,
      compiler_params=pltpu.CompilerParams(use_tc_tiling_on_sc=False),
      scratch_types=[
          pltpu.VMEM_SHARED((NACC, 16), _f32),
          pltpu.VMEM((ZROWS, 16), _f32),
          pltpu.VMEM((BLKE, 16), _f32),
          pltpu.VMEM((BLKE,), jnp.int32),
          pltpu.SemaphoreType.DMA,
      ],
  )(_deg_body)


def _scatter_body(e3_hbm, z0, z1, z2, z3,
                  s0, s1, s2, s3, acc, zbuf,
                  idx0, rows0, idx1, rows1, sem, isem):
  c = lax.axis_index("c")
  s = lax.axis_index("s")
  _fill(zbuf, 0.0, ZROWS)
  zs = (z0, z1, z2, z3)
  outs = (s0, s1, s2, s3)
  bufs = ((idx0, rows0), (idx1, rows1))
  blocks_per_tile = PAD_E // (NS * BLKE)

  for cv in (0, 1):
    @pl.when(c == cv)
    def _core(cv=cv):
      for r in (0, 1):
        ch = 2 * cv + r
        zc, oc = zs[ch], outs[ch]
        _zero_stripe(acc, s, zbuf)
        plsc.subcore_barrier()

        last_blk = PAD_E // BLKE - 1

        def load_idx(g, p):
          blk = jnp.minimum(s * blocks_per_tile + 2 * g + p, last_blk)
          return pltpu.async_copy(e3_hbm.at[blk], bufs[p][0], isem)

        load_idx(0, 0).wait()
        load_idx(0, 1).wait()

        def grp(g, carry, zc=zc):
          descs = [pltpu.async_copy(zc.at[bufs[p][0].at[0]], bufs[p][1], sem)
                   for p in (0, 1)]
          pre = []
          for p in (0, 1):
            idx, rows = bufs[p]
            descs[p].wait()
            pltpu.sync_copy(rows, acc.at[idx.at[1]], add=True)
            pre.append(load_idx(g + 1, p))
          for d in pre:
            d.wait()
          return carry

        lax.fori_loop(0, blocks_per_tile // 2, grp, None)
        plsc.subcore_barrier()
        pltpu.sync_copy(acc.at[pl.ds(s * STRIPE, STRIPE)],
                        oc.at[pl.ds(s * STRIPE, STRIPE)])
        plsc.subcore_barrier()


@functools.lru_cache(maxsize=None)
def _scatter():
  mesh = plsc.VectorSubcoreMesh(core_axis_name="c", subcore_axis_name="s")
  return functools.partial(
      pl.kernel,
      out_type=[jax.ShapeDtypeStruct((NACC, 16), _f32)] * 4,
      mesh=mesh,
      compiler_params=pltpu.CompilerParams(use_tc_tiling_on_sc=False),
      scratch_types=[
          pltpu.VMEM_SHARED((NACC, 16), _f32),
          pltpu.VMEM((ZROWS, 16), _f32),
          pltpu.VMEM((2, BLKE), jnp.int32),
          pltpu.VMEM((BLKE, 16), _f32),
          pltpu.VMEM((2, BLKE), jnp.int32),
          pltpu.VMEM((BLKE, 16), _f32),
          pltpu.SemaphoreType.DMA,
          pltpu.SemaphoreType.DMA,
      ],
  )(_scatter_body)


def _gather_body(a0, a1, a2, a3, a4, a5, a6, a7, a8, a9, ids_hbm,
                 o0, o1, o2, o3, o4, o5, o6, o7, o8, o9, idxb, rows, sem):
  c = lax.axis_index("c")
  s = lax.axis_index("s")
  wid = s * NC + c
  base = pl.multiple_of(wid * 512, 8)
  pltpu.sync_copy(ids_hbm.at[pl.ds(base, 512)], idxb)
  for a, o in ((a0, o0), (a1, o1), (a2, o2), (a3, o3), (a4, o4),
               (a5, o5), (a6, o6), (a7, o7), (a8, o8), (a9, o9)):
    pltpu.async_copy(a.at[idxb], rows, sem).wait()
    pltpu.sync_copy(rows, o.at[pl.ds(base, 512)])


@functools.lru_cache(maxsize=None)
def _gather():
  mesh = plsc.VectorSubcoreMesh(core_axis_name="c", subcore_axis_name="s")
  return functools.partial(
      pl.kernel,
      out_type=[jax.ShapeDtypeStruct((16384, 16), _f32)] * 10,
      mesh=mesh,
      compiler_params=pltpu.CompilerParams(use_tc_tiling_on_sc=False),
      scratch_types=[
          pltpu.VMEM((512,), jnp.int32),
          pltpu.VMEM((512, 16), _f32),
          pltpu.SemaphoreType.DMA,
      ],
  )(_gather_body)


_BM = 1000


def _dis(d0, d1):
  deg = d0[:, 0:1] + d1[:, 0:1] + 1.0
  return lax.rsqrt(deg)


def _k1_body(d0, d1, emb, w1, o0, o1, o2, o3):
  dis = _dis(d0[...], d1[...])
  z = jnp.dot(emb[...], w1[...], preferred_element_type=_f32) * dis
  o0[...] = z[:, 0:16]
  o1[...] = z[:, 16:32]
  o2[...] = z[:, 32:48]
  o3[...] = z[:, 48:64]


def _k2_body(d0, d1, s0, s1, s2, s3, z0, z1, z2, z3, b1, w2,
             o0, o1, o2, o3):
  dis = _dis(d0[...], d1[...])
  ssum = jnp.concatenate([s0[...] + z0[...], s1[...] + z1[...],
                          s2[...] + z2[...], s3[...] + z3[...]], axis=1)
  x1 = jax.nn.relu(dis * ssum + b1[...])
  z = jnp.dot(x1, w2[...], preferred_element_type=_f32) * dis
  o0[...] = z[:, 0:16]
  o1[...] = z[:, 16:32]
  o2[...] = z[:, 32:48]
  o3[...] = z[:, 48:64]


def _k4_body(d0, d1, s0, s1, s2, s3, z0, z1, z2, z3,
             lab, b2, wl1, bl1, wl2, bl2, wf1, bf1, wf2, bf2, out):
  dis = _dis(d0[...], d1[...])
  ssum = jnp.concatenate([s0[...] + z0[...], s1[...] + z1[...],
                          s2[...] + z2[...], s3[...] + z3[...]], axis=1)
  drug = dis * ssum + b2[...]
  lab_h = jax.nn.relu(
      jnp.dot(lab[...], wl1[...], preferred_element_type=_f32) + bl1[...])
  lab_e = jnp.dot(lab_h, wl2[...], preferred_element_type=_f32) + bl2[...]
  comb = jnp.concatenate([drug, lab_e], axis=1)
  h = jax.nn.relu(
      jnp.dot(comb, wf1[...], preferred_element_type=_f32) + bf1[...])
  o = jnp.dot(h, wf2[...], preferred_element_type=_f32) + bf2[...]
  out[...] = jax.nn.sigmoid(o)


def _row_spec(cols):
  return pl.BlockSpec((_BM, cols), lambda i: (i, 0))


def _full_spec(r, c):
  return pl.BlockSpec((r, c), lambda i: (0, 0))


def _k1(d0, d1, emb, w1):
  return pl.pallas_call(
      _k1_body,
      grid=(N // _BM,),
      in_specs=[_row_spec(16), _row_spec(16), _row_spec(32),
                _full_spec(32, 64)],
      out_specs=[_row_spec(16)] * 4,
      out_shape=[jax.ShapeDtypeStruct((N, 16), _f32)] * 4,
  )(d0, d1, emb, w1)


def _k2(d0, d1, scs, zcs, b1, w2):
  return pl.pallas_call(
      _k2_body,
      grid=(N // _BM,),
      in_specs=[_row_spec(16)] * 10 + [_full_spec(1, 64), _full_spec(64, 64)],
      out_specs=[_row_spec(16)] * 4,
      out_shape=[jax.ShapeDtypeStruct((N, 16), _f32)] * 4,
  )(d0, d1, *scs, *zcs, b1, w2)


def _k4(g10, lab, b2, wl1, bl1, wl2, bl2, wf1, bf1, wf2, bf2):
  bm = 2048
  gspec = pl.BlockSpec((bm, 16), lambda i: (i, 0))
  return pl.pallas_call(
      _k4_body,
      grid=(16384 // bm,),
      in_specs=[gspec] * 10 + [
          pl.BlockSpec((bm, 9), lambda i: (i, 0)),
          _full_spec(1, 64),
          _full_spec(9, 32), _full_spec(1, 32),
          _full_spec(32, 32), _full_spec(1, 32),
          _full_spec(96, 64), _full_spec(1, 64),
          _full_spec(64, 1), _full_spec(1, 1),
      ],
      out_specs=pl.BlockSpec((bm, 1), lambda i: (i, 0)),
      out_shape=jax.ShapeDtypeStruct((16384, 1), _f32),
  )(*g10, lab, b2, wl1, bl1, wl2, bl2, wf1, bf1, wf2, bf2)


def kernel(edge_index, drug_ids, lab_features, emb, W1, b1, W2, b2,
           Wl1, bl1, Wl2, bl2, Wf1, bf1, Wf2, bf2):
  pad = PAD_E - E
  srcp = jnp.concatenate([edge_index[0], jnp.zeros((pad,), jnp.int32)])
  dstp = jnp.concatenate([edge_index[1], jnp.full((pad,), DUMP, jnp.int32)])
  e3 = jnp.stack([srcp.reshape(-1, BLKE), dstp.reshape(-1, BLKE)], axis=1)

  d0, d1 = _deg()(dstp)
  zc1 = _k1(d0, d1, emb, W1)
  sc1 = _scatter()(e3, *zc1)
  zc2 = _k2(d0, d1, sc1, zc1, b1.reshape(1, 64), W2)
  sc2 = _scatter()(e3, *zc2)
  g10 = _gather()(d0, d1, *sc2, *zc2, drug_ids)
  o = _k4(g10, lab_features, b2.reshape(1, 64),
          Wl1, bl1.reshape(1, 32), Wl2, bl2.reshape(1, 32),
          Wf1, bf1.reshape(1, 64), Wf2, bf2.reshape(1, 1))
  return o.reshape(-1)
